# Initial kernel scaffold; baseline (speedup 1.0000x reference)
import jax
import jax.numpy as jnp
from jax.experimental import pallas as pl


def kernel(x, edge_index, W1l, b1l, W1r, W2l, b2l, W2r):
    raise NotImplementedError("write your pallas kernel here")


# speedup vs baseline: 9.0027x; 9.0027x over previous
import functools

import jax
import jax.numpy as jnp
from jax import lax
from jax.experimental import pallas as pl
from jax.experimental.pallas import tpu as pltpu
from jax.experimental.pallas import tpu_sc as plsc

N = 10000
NPAD = 10240
F = 128
H = 64
E = 320000

NC = 2
NS = 16
NW = NC * NS
B = 80
EPW = E // NW
NB = EPW // B
RPT = NPAD // NS
CH = RPT // B


def _pre_body(x_ref, wl_ref, wr_ref, b_ref, y_ref, z_ref):
    x = x_ref[...]
    y_ref[0:N] = jnp.dot(x, wl_ref[...], preferred_element_type=jnp.float32)
    y_ref[N:NPAD] = jnp.zeros((NPAD - N, H), jnp.float32)
    z_ref[...] = jnp.dot(x, wr_ref[...], preferred_element_type=jnp.float32) + b_ref[...]


def _mid_body(aggp_ref, cntp_ref, z1_ref, wl_ref, wr_ref, b_ref, y_ref, z_ref):
    agg = aggp_ref[0, 0:N] + aggp_ref[1, 0:N]
    cnt = cntp_ref[0, 0:N, 0:1] + cntp_ref[1, 0:N, 0:1]
    mean = agg / jnp.maximum(cnt, 1.0)
    h = jax.nn.sigmoid(mean + z1_ref[...])
    y_ref[0:N] = jnp.dot(h, wl_ref[...], preferred_element_type=jnp.float32)
    y_ref[N:NPAD] = jnp.zeros((NPAD - N, H), jnp.float32)
    z_ref[...] = jnp.dot(h, wr_ref[...], preferred_element_type=jnp.float32) + b_ref[...]


def _post_body(aggp_ref, cntp_ref, z2_ref, out_ref):
    agg = aggp_ref[0, 0:N] + aggp_ref[1, 0:N]
    cnt = cntp_ref[0, 0:N, 0:1] + cntp_ref[1, 0:N, 0:1]
    out_ref[...] = agg / jnp.maximum(cnt, 1.0) + z2_ref[...]


_tc_pre = pl.pallas_call(
    _pre_body,
    out_shape=[jax.ShapeDtypeStruct((NPAD, H), jnp.float32),
               jax.ShapeDtypeStruct((N, H), jnp.float32)],
)

_tc_mid = pl.pallas_call(
    _mid_body,
    out_shape=[jax.ShapeDtypeStruct((NPAD, H), jnp.float32),
               jax.ShapeDtypeStruct((N, H), jnp.float32)],
)

_tc_post = pl.pallas_call(
    _post_body,
    out_shape=jax.ShapeDtypeStruct((N, H), jnp.float32),
)


def _fill_rows(buf, ncols, val):
    v = jnp.full((16,), val, jnp.float32)

    def body(i, _):
        for j in range(ncols // 16):
            buf[i, pl.ds(j * 16, 16)] = v
        return 0

    lax.fori_loop(0, buf.shape[0], body, 0)


def _make_sc(with_cnt):
    mesh = plsc.VectorSubcoreMesh(
        core_axis_name="c", subcore_axis_name="s", num_cores=NC, num_subcores=NS)
    out_type = [jax.ShapeDtypeStruct((NC, NPAD, H), jnp.float32)]
    scratch = [
        pltpu.VMEM((NB, B), jnp.int32),
        pltpu.VMEM((NB, B), jnp.int32),
        pltpu.VMEM((B, H), jnp.float32),
        pltpu.VMEM((B, H), jnp.float32),
        pltpu.VMEM_SHARED((NPAD, H), jnp.float32),
    ]
    if with_cnt:
        out_type.append(jax.ShapeDtypeStruct((NC, NPAD, 16), jnp.float32))
        scratch += [
            pltpu.VMEM((B, 16), jnp.float32),
            pltpu.VMEM((B, 16), jnp.float32),
            pltpu.VMEM_SHARED((NPAD, 16), jnp.float32),
        ]

    def body(table, src3, dst3, *refs):
        if with_cnt:
            agg_out, cnt_out, srcv, dstv, rows, zrows, acc, ones, zc, cacc = refs
        else:
            (agg_out, srcv, dstv, rows, zrows, acc) = refs
            cnt_out = ones = zc = cacc = None
        c = lax.axis_index("c")
        s = lax.axis_index("s")
        wid = c * NS + s
        base = s * RPT

        _fill_rows(zrows, H, 0.0)
        if with_cnt:
            _fill_rows(ones, 16, 1.0)
            _fill_rows(zc, 16, 0.0)
        for k in range(CH):
            sl = pl.ds(base + k * B, B)
            pltpu.sync_copy(zrows, acc.at[sl])
            if with_cnt:
                pltpu.sync_copy(zc, cacc.at[sl])
        plsc.subcore_barrier()

        pltpu.sync_copy(src3.at[wid], srcv)
        pltpu.sync_copy(dst3.at[wid], dstv)

        def step(j, _):
            pltpu.sync_copy(table.at[srcv.at[j]], rows)
            pltpu.sync_copy(rows, acc.at[dstv.at[j]], add=True)
            if with_cnt:
                pltpu.sync_copy(ones, cacc.at[dstv.at[j]], add=True)
            return 0

        lax.fori_loop(0, NB, step, 0)
        plsc.subcore_barrier()

        for k in range(CH):
            sl = pl.ds(base + k * B, B)
            pltpu.sync_copy(acc.at[sl], zrows)
            pltpu.sync_copy(zrows, agg_out.at[c, sl])
            if with_cnt:
                pltpu.sync_copy(cacc.at[sl], ones)
                pltpu.sync_copy(ones, cnt_out.at[c, sl])

    return pl.kernel(
        body, out_type=out_type, mesh=mesh, scratch_types=scratch,
        compiler_params=pltpu.CompilerParams(use_tc_tiling_on_sc=False))


_sc_l1 = _make_sc(with_cnt=True)
_sc_l2 = _make_sc(with_cnt=False)


@jax.jit
def kernel(x, edge_index, W1l, b1l, W1r, W2l, b2l, W2r):
    e = edge_index.astype(jnp.int32)
    src3 = e[0].reshape(NW, NB, B)
    dst3 = e[1].reshape(NW, NB, B)

    y1, z1 = _tc_pre(x, W1l, W1r, b1l.reshape(1, H))
    agg1p, cntp = _sc_l1(y1, src3, dst3)
    y2, z2 = _tc_mid(agg1p, cntp, z1, W2l, W2r, b2l.reshape(1, H))
    [agg2p] = _sc_l2(y2, src3, dst3)
    return _tc_post(agg2p, cntp, z2)


# speedup vs baseline: 13.3572x; 1.4837x over previous
import functools

import jax
import jax.numpy as jnp
from jax import lax
from jax.experimental import pallas as pl
from jax.experimental.pallas import tpu as pltpu
from jax.experimental.pallas import tpu_sc as plsc

N = 10000
NPAD = 10240
F = 128
H = 64
E = 320000

NC = 2
NS = 16
NW = NC * NS
B = 80
EPW = E // NW
NB = EPW // B
RPT = NPAD // NS
CH = RPT // B


def _pre_body(x_ref, wl_ref, wr_ref, b_ref, y_ref, z_ref):
    x = x_ref[...]
    y_ref[0:N] = jnp.dot(x, wl_ref[...], preferred_element_type=jnp.float32)
    y_ref[N:NPAD] = jnp.zeros((NPAD - N, H), jnp.float32)
    z_ref[...] = jnp.dot(x, wr_ref[...], preferred_element_type=jnp.float32) + b_ref[...]


def _mid_body(aggp_ref, cntp_ref, z1_ref, wl_ref, wr_ref, b_ref, y_ref, z_ref):
    agg = aggp_ref[0, 0:N] + aggp_ref[1, 0:N]
    cnt = cntp_ref[0, 0:N, 0:1] + cntp_ref[1, 0:N, 0:1]
    mean = agg / jnp.maximum(cnt, 1.0)
    h = jax.nn.sigmoid(mean + z1_ref[...])
    y_ref[0:N] = jnp.dot(h, wl_ref[...], preferred_element_type=jnp.float32)
    y_ref[N:NPAD] = jnp.zeros((NPAD - N, H), jnp.float32)
    z_ref[...] = jnp.dot(h, wr_ref[...], preferred_element_type=jnp.float32) + b_ref[...]


def _post_body(aggp_ref, cntp_ref, z2_ref, out_ref):
    agg = aggp_ref[0, 0:N] + aggp_ref[1, 0:N]
    cnt = cntp_ref[0, 0:N, 0:1] + cntp_ref[1, 0:N, 0:1]
    out_ref[...] = agg / jnp.maximum(cnt, 1.0) + z2_ref[...]


_tc_pre = pl.pallas_call(
    _pre_body,
    out_shape=[jax.ShapeDtypeStruct((NPAD, H), jnp.float32),
               jax.ShapeDtypeStruct((N, H), jnp.float32)],
)

_tc_mid = pl.pallas_call(
    _mid_body,
    out_shape=[jax.ShapeDtypeStruct((NPAD, H), jnp.float32),
               jax.ShapeDtypeStruct((N, H), jnp.float32)],
)

_tc_post = pl.pallas_call(
    _post_body,
    out_shape=jax.ShapeDtypeStruct((N, H), jnp.float32),
)


def _fill_rows(buf, ncols, val):
    v = jnp.full((16,), val, jnp.float32)

    def body(i, _):
        for j in range(ncols // 16):
            buf[i, pl.ds(j * 16, 16)] = v
        return 0

    lax.fori_loop(0, buf.shape[0], body, 0)


def _make_sc(with_cnt):
    mesh = plsc.VectorSubcoreMesh(
        core_axis_name="c", subcore_axis_name="s", num_cores=NC, num_subcores=NS)
    out_type = [jax.ShapeDtypeStruct((NC, NPAD, H), jnp.float32)]
    scratch = [
        pltpu.VMEM((NB, B), jnp.int32),
        pltpu.VMEM((NB, B), jnp.int32),
        pltpu.VMEM((B, H), jnp.float32),
        pltpu.VMEM((B, H), jnp.float32),
        pltpu.VMEM((B, H), jnp.float32),
        pltpu.VMEM_SHARED((NPAD, H), jnp.float32),
        pltpu.SemaphoreType.DMA,
        pltpu.SemaphoreType.DMA,
    ]
    if with_cnt:
        out_type.append(jax.ShapeDtypeStruct((NC, NPAD, 16), jnp.float32))
        scratch += [
            pltpu.VMEM((B, 16), jnp.float32),
            pltpu.VMEM((B, 16), jnp.float32),
            pltpu.VMEM_SHARED((NPAD, 16), jnp.float32),
        ]

    def body(table, src3, dst3, *refs):
        if with_cnt:
            (agg_out, cnt_out, srcv, dstv, rows0, rows1, zrows, acc, sem0, sem1,
             ones, zc, cacc) = refs
        else:
            (agg_out, srcv, dstv, rows0, rows1, zrows, acc, sem0, sem1) = refs
            cnt_out = ones = zc = cacc = None
        c = lax.axis_index("c")
        s = lax.axis_index("s")
        wid = c * NS + s
        base = s * RPT

        _fill_rows(zrows, H, 0.0)
        if with_cnt:
            _fill_rows(ones, 16, 1.0)
            _fill_rows(zc, 16, 0.0)
        for k in range(CH):
            sl = pl.ds(base + k * B, B)
            pltpu.sync_copy(zrows, acc.at[sl])
            if with_cnt:
                pltpu.sync_copy(zc, cacc.at[sl])
        plsc.subcore_barrier()

        pltpu.sync_copy(src3.at[wid], srcv)
        pltpu.sync_copy(dst3.at[wid], dstv)

        def scat(rbuf, j):
            pltpu.sync_copy(rbuf, acc.at[dstv.at[j]], add=True)
            if with_cnt:
                pltpu.sync_copy(ones, cacc.at[dstv.at[j]], add=True)

        pltpu.async_copy(table.at[srcv.at[0]], rows0, sem0)

        def step(t, _):
            pltpu.async_copy(table.at[srcv.at[2 * t + 1]], rows1, sem1)
            pltpu.make_async_copy(table.at[srcv.at[2 * t]], rows0, sem0).wait()
            scat(rows0, 2 * t)
            pltpu.async_copy(table.at[srcv.at[2 * t + 2]], rows0, sem0)
            pltpu.make_async_copy(table.at[srcv.at[2 * t + 1]], rows1, sem1).wait()
            scat(rows1, 2 * t + 1)
            return 0

        lax.fori_loop(0, (NB - 1) // 2, step, 0)
        pltpu.make_async_copy(table.at[srcv.at[NB - 1]], rows0, sem0).wait()
        scat(rows0, NB - 1)
        plsc.subcore_barrier()

        for k in range(CH):
            sl = pl.ds(base + k * B, B)
            pltpu.sync_copy(acc.at[sl], zrows)
            pltpu.sync_copy(zrows, agg_out.at[c, sl])
            if with_cnt:
                pltpu.sync_copy(cacc.at[sl], ones)
                pltpu.sync_copy(ones, cnt_out.at[c, sl])

    return pl.kernel(
        body, out_type=out_type, mesh=mesh, scratch_types=scratch,
        compiler_params=pltpu.CompilerParams(use_tc_tiling_on_sc=False))


_sc_l1 = _make_sc(with_cnt=True)
_sc_l2 = _make_sc(with_cnt=False)


@jax.jit
def kernel(x, edge_index, W1l, b1l, W1r, W2l, b2l, W2r):
    e = edge_index.astype(jnp.int32)
    src3 = e[0].reshape(NW, NB, B)
    dst3 = e[1].reshape(NW, NB, B)

    y1, z1 = _tc_pre(x, W1l, W1r, b1l.reshape(1, H))
    agg1p, cntp = _sc_l1(y1, src3, dst3)
    y2, z2 = _tc_mid(agg1p, cntp, z1, W2l, W2r, b2l.reshape(1, H))
    [agg2p] = _sc_l2(y2, src3, dst3)
    return _tc_post(agg2p, cntp, z2)


# speedup vs baseline: 16.7835x; 1.2565x over previous
import functools

import jax
import jax.numpy as jnp
from jax import lax
from jax.experimental import pallas as pl
from jax.experimental.pallas import tpu as pltpu
from jax.experimental.pallas import tpu_sc as plsc

N = 10000
NPAD = 10240
F = 128
H = 64
E = 320000

NC = 2
NS = 16
NW = NC * NS
B = 80
EPW = E // NW
NB = EPW // B
RPT = NPAD // NS
CH = RPT // B
GB = 5


def _pre_body(x_ref, wl_ref, wr_ref, b_ref, y_ref, z_ref):
    x = x_ref[...]
    y_ref[0:N] = jnp.dot(x, wl_ref[...], preferred_element_type=jnp.float32)
    y_ref[N:NPAD] = jnp.zeros((NPAD - N, H), jnp.float32)
    z_ref[...] = jnp.dot(x, wr_ref[...], preferred_element_type=jnp.float32) + b_ref[...]


def _mid_body(aggp_ref, cntp_ref, z1_ref, wl_ref, wr_ref, b_ref, y_ref, z_ref):
    agg = aggp_ref[0, 0:N] + aggp_ref[1, 0:N]
    cnt = cntp_ref[0, 0:N, 0:1] + cntp_ref[1, 0:N, 0:1]
    mean = agg / jnp.maximum(cnt, 1.0)
    h = jax.nn.sigmoid(mean + z1_ref[...])
    y_ref[0:N] = jnp.dot(h, wl_ref[...], preferred_element_type=jnp.float32)
    y_ref[N:NPAD] = jnp.zeros((NPAD - N, H), jnp.float32)
    z_ref[...] = jnp.dot(h, wr_ref[...], preferred_element_type=jnp.float32) + b_ref[...]


def _post_body(aggp_ref, cntp_ref, z2_ref, out_ref):
    agg = aggp_ref[0, 0:N] + aggp_ref[1, 0:N]
    cnt = cntp_ref[0, 0:N, 0:1] + cntp_ref[1, 0:N, 0:1]
    out_ref[...] = agg / jnp.maximum(cnt, 1.0) + z2_ref[...]


_tc_pre = pl.pallas_call(
    _pre_body,
    out_shape=[jax.ShapeDtypeStruct((NPAD, H), jnp.float32),
               jax.ShapeDtypeStruct((N, H), jnp.float32)],
)

_tc_mid = pl.pallas_call(
    _mid_body,
    out_shape=[jax.ShapeDtypeStruct((NPAD, H), jnp.float32),
               jax.ShapeDtypeStruct((N, H), jnp.float32)],
)

_tc_post = pl.pallas_call(
    _post_body,
    out_shape=jax.ShapeDtypeStruct((N, H), jnp.float32),
)


def _fill_rows(buf, ncols, val):
    v = jnp.full((16,), val, jnp.float32)

    def body(i, _):
        for j in range(ncols // 16):
            buf[i, pl.ds(j * 16, 16)] = v
        return 0

    lax.fori_loop(0, buf.shape[0], body, 0)


def _make_sc(with_cnt):
    mesh = plsc.VectorSubcoreMesh(
        core_axis_name="c", subcore_axis_name="s", num_cores=NC, num_subcores=NS)
    out_type = [jax.ShapeDtypeStruct((NC, NPAD, H), jnp.float32)]
    scratch = (
        [pltpu.VMEM((NB, B), jnp.int32),
         pltpu.VMEM((NB, B), jnp.int32)]
        + [pltpu.VMEM((B, H), jnp.float32)] * GB
        + [pltpu.VMEM((B, H), jnp.float32),
           pltpu.VMEM_SHARED((NPAD, H), jnp.float32)]
        + [pltpu.SemaphoreType.DMA] * GB
        + [pltpu.SemaphoreType.DMA] * GB
    )
    if with_cnt:
        out_type.append(jax.ShapeDtypeStruct((NC, NPAD, 16), jnp.float32))
        scratch = scratch + [
            pltpu.VMEM((B, 16), jnp.float32),
            pltpu.VMEM((B, 16), jnp.float32),
            pltpu.VMEM_SHARED((NPAD, 16), jnp.float32),
            pltpu.SemaphoreType.DMA,
        ]

    def body(table, src3, dst3, *refs):
        if with_cnt:
            (agg_out, cnt_out, srcv, dstv, *rest) = refs
            rows = rest[:GB]
            zrows, acc = rest[GB], rest[GB + 1]
            gsem = rest[GB + 2:2 * GB + 2]
            ssem = rest[2 * GB + 2:3 * GB + 2]
            ones, zc, cacc, csem = rest[3 * GB + 2:]
        else:
            (agg_out, srcv, dstv, *rest) = refs
            rows = rest[:GB]
            zrows, acc = rest[GB], rest[GB + 1]
            gsem = rest[GB + 2:2 * GB + 2]
            ssem = rest[2 * GB + 2:3 * GB + 2]
            cnt_out = ones = zc = cacc = csem = None
        c = lax.axis_index("c")
        s = lax.axis_index("s")
        wid = c * NS + s
        base = s * RPT

        _fill_rows(zrows, H, 0.0)
        if with_cnt:
            _fill_rows(ones, 16, 1.0)
            _fill_rows(zc, 16, 0.0)
        for k in range(CH):
            sl = pl.ds(base + k * B, B)
            pltpu.sync_copy(zrows, acc.at[sl])
            if with_cnt:
                pltpu.sync_copy(zc, cacc.at[sl])
        plsc.subcore_barrier()

        pltpu.sync_copy(src3.at[wid], srcv)
        pltpu.sync_copy(dst3.at[wid], dstv)

        LEAD = GB - 2

        def wait_g(b, j):
            pltpu.make_async_copy(table.at[srcv.at[j]], rows[b], gsem[b]).wait()

        def wait_s(b):
            pltpu.make_async_copy(rows[b], acc.at[dstv.at[0]], ssem[b]).wait()

        def issue(j, b, do_swait, do_gather):
            bw = (b + LEAD) % GB
            if do_swait:
                wait_s(bw)
            if do_gather:
                pltpu.async_copy(table.at[srcv.at[j + LEAD]], rows[bw], gsem[bw])
            wait_g(b, j)
            pltpu.async_copy(rows[b], acc.at[dstv.at[j]], ssem[b], add=True)
            if with_cnt:
                pltpu.async_copy(ones, cacc.at[dstv.at[j]], csem, add=True)

        for b in range(LEAD):
            pltpu.async_copy(table.at[srcv.at[b]], rows[b], gsem[b])
        for b in range(GB):
            issue(b, b, do_swait=(b >= GB - LEAD), do_gather=True)

        def outer(t, _):
            for b in range(GB):
                issue(GB * t + b, b, do_swait=True, do_gather=True)
            return 0

        lax.fori_loop(1, NB // GB - 1, outer, 0)
        for b in range(GB):
            j = NB - GB + b
            issue(j, b, do_swait=True, do_gather=(j + LEAD < NB))
        for b in range(LEAD, GB):
            wait_s(b)
        if with_cnt:

            def cdrain(i, _):
                pltpu.make_async_copy(ones, cacc.at[dstv.at[0]], csem).wait()
                return 0

            lax.fori_loop(0, NB, cdrain, 0)
        plsc.subcore_barrier()

        for k in range(CH):
            sl = pl.ds(base + k * B, B)
            pltpu.sync_copy(acc.at[sl], zrows)
            pltpu.sync_copy(zrows, agg_out.at[c, sl])
            if with_cnt:
                pltpu.sync_copy(cacc.at[sl], ones)
                pltpu.sync_copy(ones, cnt_out.at[c, sl])

    return pl.kernel(
        body, out_type=out_type, mesh=mesh, scratch_types=scratch,
        compiler_params=pltpu.CompilerParams(use_tc_tiling_on_sc=False))


_sc_l1 = _make_sc(with_cnt=True)
_sc_l2 = _make_sc(with_cnt=False)


@jax.jit
def kernel(x, edge_index, W1l, b1l, W1r, W2l, b2l, W2r):
    e = edge_index.astype(jnp.int32)
    src3 = e[0].reshape(NW, NB, B)
    dst3 = e[1].reshape(NW, NB, B)

    y1, z1 = _tc_pre(x, W1l, W1r, b1l.reshape(1, H))
    agg1p, cntp = _sc_l1(y1, src3, dst3)
    y2, z2 = _tc_mid(agg1p, cntp, z1, W2l, W2r, b2l.reshape(1, H))
    [agg2p] = _sc_l2(y2, src3, dst3)
    return _tc_post(agg2p, cntp, z2)


# speedup vs baseline: 19.2289x; 1.1457x over previous
import functools

import jax
import jax.numpy as jnp
from jax import lax
from jax.experimental import pallas as pl
from jax.experimental.pallas import tpu as pltpu
from jax.experimental.pallas import tpu_sc as plsc

N = 10000
NPAD = 10240
F = 128
H = 64
E = 320000

NC = 2
NS = 16
NW = NC * NS
B = 80
EPW = E // NW
NB = EPW // B
RPT = NPAD // NS
CH = RPT // B
GB = 5


def _pre_body(x_ref, wl_ref, wr_ref, b_ref, y_ref, z_ref):
    x = x_ref[...]
    y_ref[0:N] = jnp.dot(x, wl_ref[...], preferred_element_type=jnp.float32)
    y_ref[N:NPAD] = jnp.zeros((NPAD - N, H), jnp.float32)
    z_ref[...] = jnp.dot(x, wr_ref[...], preferred_element_type=jnp.float32) + b_ref[...]


def _mid_body(aggp_ref, cntp_ref, z1_ref, wl_ref, wr_ref, b_ref, y_ref, z_ref):
    agg = aggp_ref[0:N, 0:H] + aggp_ref[0:N, H:2 * H]
    cnt = cntp_ref[0, 0:N, 0:1] + cntp_ref[1, 0:N, 0:1]
    mean = agg / jnp.maximum(cnt, 1.0)
    h = jax.nn.sigmoid(mean + z1_ref[...])
    y_ref[0:N] = jnp.dot(h, wl_ref[...], preferred_element_type=jnp.float32)
    y_ref[N:NPAD] = jnp.zeros((NPAD - N, H), jnp.float32)
    z_ref[...] = jnp.dot(h, wr_ref[...], preferred_element_type=jnp.float32) + b_ref[...]


def _post_body(aggp_ref, cntp_ref, z2_ref, out_ref):
    agg = aggp_ref[0:N, 0:H] + aggp_ref[0:N, H:2 * H]
    cnt = cntp_ref[0, 0:N, 0:1] + cntp_ref[1, 0:N, 0:1]
    out_ref[...] = agg / jnp.maximum(cnt, 1.0) + z2_ref[...]


_tc_pre = pl.pallas_call(
    _pre_body,
    out_shape=[jax.ShapeDtypeStruct((NPAD, H), jnp.float32),
               jax.ShapeDtypeStruct((N, H), jnp.float32)],
)

_tc_mid = pl.pallas_call(
    _mid_body,
    out_shape=[jax.ShapeDtypeStruct((NPAD, H), jnp.float32),
               jax.ShapeDtypeStruct((N, H), jnp.float32)],
)

_tc_post = pl.pallas_call(
    _post_body,
    out_shape=jax.ShapeDtypeStruct((N, H), jnp.float32),
)


def _fill_rows(buf, ncols, val):
    v = jnp.full((16,), val, jnp.float32)

    def body(i, _):
        for j in range(ncols // 16):
            buf[i, pl.ds(j * 16, 16)] = v
        return 0

    lax.fori_loop(0, buf.shape[0], body, 0)


def _make_sc(with_cnt):
    mesh = plsc.VectorSubcoreMesh(
        core_axis_name="c", subcore_axis_name="s", num_cores=NC, num_subcores=NS)
    out_type = [jax.ShapeDtypeStruct((NPAD, NC * H), jnp.float32)]
    scratch = (
        [pltpu.VMEM((NB, B), jnp.int32),
         pltpu.VMEM((NB, B), jnp.int32)]
        + [pltpu.VMEM((B, H), jnp.float32)] * GB
        + [pltpu.VMEM((B, H), jnp.float32),
           pltpu.VMEM_SHARED((NPAD, H), jnp.float32)]
        + [pltpu.SemaphoreType.DMA] * GB
        + [pltpu.SemaphoreType.DMA] * GB
    )
    if with_cnt:
        out_type.append(jax.ShapeDtypeStruct((NC, NPAD, 16), jnp.float32))
        scratch = scratch + [
            pltpu.VMEM((B, 16), jnp.float32),
            pltpu.VMEM((B, 16), jnp.float32),
            pltpu.VMEM_SHARED((NPAD, 16), jnp.float32),
            pltpu.SemaphoreType.DMA,
        ]

    def body(table, idx3, *refs):
        if with_cnt:
            (agg_out, cnt_out, srcv, dstv, *rest) = refs
            rows = rest[:GB]
            zrows, acc = rest[GB], rest[GB + 1]
            gsem = rest[GB + 2:2 * GB + 2]
            ssem = rest[2 * GB + 2:3 * GB + 2]
            ones, zc, cacc, csem = rest[3 * GB + 2:]
        else:
            (agg_out, srcv, dstv, *rest) = refs
            rows = rest[:GB]
            zrows, acc = rest[GB], rest[GB + 1]
            gsem = rest[GB + 2:2 * GB + 2]
            ssem = rest[2 * GB + 2:3 * GB + 2]
            cnt_out = ones = zc = cacc = csem = None
        c = lax.axis_index("c")
        s = lax.axis_index("s")
        wid = c * NS + s
        base = s * RPT

        _fill_rows(zrows, H, 0.0)
        if with_cnt:
            _fill_rows(ones, 16, 1.0)
            _fill_rows(zc, 16, 0.0)
        for k in range(CH):
            sl = pl.ds(base + k * B, B)
            pltpu.sync_copy(zrows, acc.at[sl])
            if with_cnt:
                pltpu.sync_copy(zc, cacc.at[sl])
        plsc.subcore_barrier()

        pltpu.sync_copy(idx3.at[0, wid], srcv)
        pltpu.sync_copy(idx3.at[1, wid], dstv)

        LEAD = GB - 2

        def wait_g(b, j):
            pltpu.make_async_copy(table.at[srcv.at[j]], rows[b], gsem[b]).wait()

        def wait_s(b):
            pltpu.make_async_copy(rows[b], acc.at[dstv.at[0]], ssem[b]).wait()

        def issue(j, b, do_swait, do_gather):
            bw = (b + LEAD) % GB
            if do_swait:
                wait_s(bw)
            if do_gather:
                pltpu.async_copy(table.at[srcv.at[j + LEAD]], rows[bw], gsem[bw])
            wait_g(b, j)
            pltpu.async_copy(rows[b], acc.at[dstv.at[j]], ssem[b], add=True)
            if with_cnt:
                pltpu.async_copy(ones, cacc.at[dstv.at[j]], csem, add=True)

        for b in range(LEAD):
            pltpu.async_copy(table.at[srcv.at[b]], rows[b], gsem[b])
        for b in range(GB):
            issue(b, b, do_swait=(b >= GB - LEAD), do_gather=True)

        def outer(t, _):
            for b in range(GB):
                issue(GB * t + b, b, do_swait=True, do_gather=True)
            return 0

        lax.fori_loop(1, NB // GB - 1, outer, 0)
        for b in range(GB):
            j = NB - GB + b
            issue(j, b, do_swait=True, do_gather=(j + LEAD < NB))
        for b in range(LEAD, GB):
            wait_s(b)
        if with_cnt:

            def cdrain(i, _):
                pltpu.make_async_copy(ones, cacc.at[dstv.at[0]], csem).wait()
                return 0

            lax.fori_loop(0, NB, cdrain, 0)
        plsc.subcore_barrier()

        for k in range(CH):
            sl = pl.ds(base + k * B, B)
            pltpu.sync_copy(acc.at[sl], zrows)
            pltpu.sync_copy(zrows, agg_out.at[sl, pl.ds(c * H, H)])
            if with_cnt:
                pltpu.sync_copy(cacc.at[sl], ones)
                pltpu.sync_copy(ones, cnt_out.at[c, sl])

    return pl.kernel(
        body, out_type=out_type, mesh=mesh, scratch_types=scratch,
        compiler_params=pltpu.CompilerParams(use_tc_tiling_on_sc=False))


_sc_l1 = _make_sc(with_cnt=True)
_sc_l2 = _make_sc(with_cnt=False)


@jax.jit
def kernel(x, edge_index, W1l, b1l, W1r, W2l, b2l, W2r):
    idx3 = edge_index.astype(jnp.int32).reshape(2, NW, NB, B)

    y1, z1 = _tc_pre(x, W1l, W1r, b1l.reshape(1, H))
    agg1p, cntp = _sc_l1(y1, idx3)
    y2, z2 = _tc_mid(agg1p, cntp, z1, W2l, W2r, b2l.reshape(1, H))
    [agg2p] = _sc_l2(y2, idx3)
    return _tc_post(agg2p, cntp, z2)
